# Initial kernel scaffold; baseline (speedup 1.0000x reference)
import jax
import jax.numpy as jnp
from jax.experimental import pallas as pl


def kernel(x_fine, x_coarse, x_global, edge_index_fine_coarse, edge_index_coarse_global, params):
    raise NotImplementedError("write your pallas kernel here")


# speedup vs baseline: 1.0768x; 1.0768x over previous
import functools

import jax
import jax.numpy as jnp
from jax import lax
from jax.experimental import pallas as pl
from jax.experimental.pallas import tpu as pltpu

F32 = jnp.float32
NEG = -1e30


def _zspec(shapes):
    def mk(s):
        return pl.BlockSpec(s, lambda s=s: (0,) * len(s))
    return [mk(s) for s in shapes]


def _ln(x, w, b):
    mu = jnp.mean(x, axis=-1, keepdims=True)
    var = jnp.var(x, axis=-1, keepdims=True)
    return (x - mu) * lax.rsqrt(var + 1e-5) * w + b


def _dot(a, b):
    return jnp.dot(a, b, preferred_element_type=F32)


def _fine_front_body(x_ref, lnw_ref, lnb_ref, w1_ref, b1_ref, w2_ref, b2_ref,
                     wsrc_ref, asrc_ref, xs_ref, als_ref):
    x = _ln(x_ref[...], lnw_ref[...], lnb_ref[...])
    h = jnp.maximum(_dot(x, w1_ref[...].T) + b1_ref[...], 0.0)
    xf = _dot(h, w2_ref[...].T) + b2_ref[...]
    xs = _dot(xf, wsrc_ref[...].T)
    xs_ref[...] = xs
    als_ref[...] = _dot(xs, asrc_ref[...])


def _fine_front(x_fine, p):
    N, BF = 10000, 1000
    H1, C1 = 8, 64
    a = p['g1_asrc']
    eye = jnp.eye(H1, dtype=F32)
    A = (a[:, :, None] * eye[:, None, :]).reshape(H1 * C1, H1)
    grid = (N // BF,)
    xs, als = pl.pallas_call(
        _fine_front_body,
        grid=grid,
        in_specs=[
            pl.BlockSpec((BF, 128), lambda i: (i, 0)),
            pl.BlockSpec((128,), lambda i: (0,)),
            pl.BlockSpec((128,), lambda i: (0,)),
            pl.BlockSpec((256, 128), lambda i: (0, 0)),
            pl.BlockSpec((256,), lambda i: (0,)),
            pl.BlockSpec((128, 256), lambda i: (0, 0)),
            pl.BlockSpec((128,), lambda i: (0,)),
            pl.BlockSpec((512, 128), lambda i: (0, 0)),
            pl.BlockSpec((512, 8), lambda i: (0, 0)),
        ],
        out_specs=[
            pl.BlockSpec((BF, 512), lambda i: (i, 0)),
            pl.BlockSpec((BF, 8), lambda i: (i, 0)),
        ],
        out_shape=[
            jax.ShapeDtypeStruct((N, 512), F32),
            jax.ShapeDtypeStruct((N, 8), F32),
        ],
    )(x_fine, p['ln_fine_w'], p['ln_fine_b'], p['ip_w1'], p['ip_b1'],
      p['ip_w2'], p['ip_b2'], p['g1_wsrc'], A)
    return xs, als


def _coarse_front_body(x_ref, lnw_ref, lnb_ref, w1_ref, b1_ref, w2_ref, b2_ref,
                       wdst_ref, adst_ref, ald_ref):
    x = _ln(x_ref[...], lnw_ref[...], lnb_ref[...])
    h = jnp.maximum(_dot(x, w1_ref[...].T) + b1_ref[...], 0.0)
    xc = _dot(h, w2_ref[...].T) + b2_ref[...]
    xd = _dot(xc, wdst_ref[...].T)
    ald_ref[...] = _dot(xd, adst_ref[...])


def _coarse_front(x_coarse, p):
    H1, C1 = 8, 64
    a = p['g1_adst']
    eye = jnp.eye(H1, dtype=F32)
    A = (a[:, :, None] * eye[:, None, :]).reshape(H1 * C1, H1)
    ald = pl.pallas_call(
        _coarse_front_body,
        in_specs=_zspec([(2000, 128), (128,), (128,), (256, 128), (256,),
                         (128, 256), (128,), (512, 128), (512, 8)]),
        out_specs=pl.BlockSpec((2000, 8), lambda: (0, 0)),
        out_shape=jax.ShapeDtypeStruct((2000, 8), F32),
    )(x_coarse, p['ln_fine_w'], p['ln_fine_b'], p['ip_w1'], p['ip_b1'],
      p['ip_w2'], p['ip_b2'], p['g1_wdst'], A)
    return ald


def _ff_body(x_ref, lnw_ref, lnb_ref, w1_ref, b1_ref, w2_ref, b2_ref,
             fw1_ref, fb1_ref, fw2_ref, fb2_ref, o_ref):
    x = _ln(x_ref[...], lnw_ref[...], lnb_ref[...])
    h = jnp.maximum(_dot(x, w1_ref[...].T) + b1_ref[...], 0.0)
    xf = _dot(h, w2_ref[...].T) + b2_ref[...]
    g = jnp.maximum(_dot(xf, fw1_ref[...].T) + fb1_ref[...], 0.0)
    o_ref[...] = _dot(g, fw2_ref[...].T) + fb2_ref[...]


def _fine_out(x_fine, p):
    N, BF = 10000, 1000
    out = pl.pallas_call(
        _ff_body,
        grid=(N // BF,),
        in_specs=[
            pl.BlockSpec((BF, 128), lambda i: (i, 0)),
            pl.BlockSpec((128,), lambda i: (0,)),
            pl.BlockSpec((128,), lambda i: (0,)),
            pl.BlockSpec((256, 128), lambda i: (0, 0)),
            pl.BlockSpec((256,), lambda i: (0,)),
            pl.BlockSpec((128, 256), lambda i: (0, 0)),
            pl.BlockSpec((128,), lambda i: (0,)),
            pl.BlockSpec((2048, 128), lambda i: (0, 0)),
            pl.BlockSpec((2048,), lambda i: (0,)),
            pl.BlockSpec((2048, 2048), lambda i: (0, 0)),
            pl.BlockSpec((2048,), lambda i: (0,)),
        ],
        out_specs=pl.BlockSpec((BF, 2048), lambda i: (i, 0)),
        out_shape=jax.ShapeDtypeStruct((N, 2048), F32),
    )(x_fine, p['ln_fine_w'], p['ln_fine_b'], p['ip_w1'], p['ip_b1'],
      p['ip_w2'], p['ip_b2'], p['fp_w1'], p['fp_b1'], p['fp_w2'], p['fp_b2'])
    return out


def _gat1_edges(xs, als, ald, src, dst, valid, M):
    e = jax.nn.leaky_relu(als[src] + ald[dst], 0.2)
    ex = jnp.where(valid[:, None], jnp.exp(e - M[None, :]), 0.0)
    den = jax.ops.segment_sum(ex, dst, num_segments=2000)
    xsr = xs.reshape(10000, 8, 64)
    U = jax.ops.segment_sum(xsr[src] * ex[:, :, None], dst, num_segments=2000)
    return U.reshape(2000, 512), den


def _tail_a_body(u_ref, den_ref, b_ref, lnw_ref, lnb_ref, wsrc_ref, asrc_ref,
                 xc2_ref, xs2_ref, als2_ref):
    den = den_ref[...]
    inv = 1.0 / (den + 1e-16)
    u = u_ref[...].reshape(2000, 8, 64) * inv[:, :, None]
    out1 = u.reshape(2000, 512) + b_ref[...]
    xc2 = jnp.maximum(_ln(out1, lnw_ref[...], lnb_ref[...]), 0.0)
    xc2_ref[...] = xc2
    xs2 = _dot(xc2, wsrc_ref[...].T)
    xs2_ref[...] = xs2
    als2_ref[...] = _dot(xs2, asrc_ref[...])


def _tail_a(U, den, p):
    H2, C2 = 4, 128
    a = p['g2_asrc']
    eye = jnp.eye(H2, dtype=F32)
    A = (a[:, :, None] * eye[:, None, :]).reshape(H2 * C2, H2)
    xc2, xs2, als2 = pl.pallas_call(
        _tail_a_body,
        in_specs=_zspec([(2000, 512), (2000, 8), (512,), (512,), (512,),
                         (512, 512), (512, 4)]),
        out_specs=_zspec([(2000, 512), (2000, 512), (2000, 4)]),
        out_shape=[jax.ShapeDtypeStruct((2000, 512), F32),
                   jax.ShapeDtypeStruct((2000, 512), F32),
                   jax.ShapeDtypeStruct((2000, 4), F32)],
    )(U, den, p['g1_b'], p['ln_coarse_w'], p['ln_coarse_b'], p['g2_wsrc'], A)
    return xc2, xs2, als2


E2P = 2048


def _tail_b_body(xg_ref, lnw_ref, lnb_ref, wdst_ref, adst_ref,
                 xs2_ref, als2_ref, src_ref, dst_ref, val_ref, b2_ref,
                 gw_ref, gb_ref, xgout_ref):
    xg = _ln(xg_ref[...], lnw_ref[...], lnb_ref[...])
    xd2 = _dot(xg, wdst_ref[...].T)
    ald2 = _dot(xd2, adst_ref[...])
    src = src_ref[0]
    dst = dst_ref[0]
    val = val_ref[0]
    s_iota = lax.broadcasted_iota(jnp.int32, (E2P, 2000), 1)
    S = (src[:, None] == s_iota).astype(F32)
    d_iota = lax.broadcasted_iota(jnp.int32, (E2P, 32), 1)
    D = (dst[:, None] == d_iota).astype(F32)[:, :20]
    als2e = _dot(S, als2_ref[...])
    ald2e = _dot(D, ald2)
    s_ = als2e + ald2e
    e = jnp.where(s_ >= 0, s_, 0.2 * s_)
    e = jnp.where(val[:, None] > 0, e, NEG)
    emax = jnp.max(jnp.where(D.T[:, :, None] > 0, e[None, :, :], NEG), axis=1)
    emax = jnp.maximum(emax, NEG * 0.5)
    ex = jnp.where(val[:, None] > 0, jnp.exp(e - _dot(D, emax)), 0.0)
    den = _dot(D.T, ex)
    xs2e = _dot(S, xs2_ref[...])
    exb = jnp.repeat(ex, 128, axis=1)
    U2 = _dot(D.T, exb * xs2e)
    inv = 1.0 / (den + 1e-16)
    out2 = U2.reshape(20, 4, 128) * inv[:, :, None]
    xg2 = out2.reshape(20, 512) + b2_ref[...] + xg
    xgout_ref[...] = _dot(xg2, gw_ref[...].T) + gb_ref[...]


def _tail_b(x_global, xs2, als2, src2, dst2, val2, p):
    xgout = pl.pallas_call(
        _tail_b_body,
        in_specs=_zspec([(20, 512), (512,), (512,), (512, 512), (512, 4),
                         (2000, 512), (2000, 4), (1, E2P), (1, E2P), (1, E2P),
                         (512,), (2048, 512), (2048,)]),
        out_specs=pl.BlockSpec((20, 2048), lambda: (0, 0)),
        out_shape=jax.ShapeDtypeStruct((20, 2048), F32),
    )(x_global, p['ln_global_w'], p['ln_global_b'], p['g2_wdst'],
      _blockdiag(p['g2_adst'], 4, 128), xs2, als2,
      src2[None, :], dst2[None, :], val2[None, :],
      p['g2_b'], p['glob_w'], p['glob_b'])
    return xgout


def _blockdiag(a, H, C):
    eye = jnp.eye(H, dtype=F32)
    return (a[:, :, None] * eye[:, None, :]).reshape(H * C, H)


def _crs_body(x_ref, w_ref, b_ref, o_ref):
    o_ref[...] = _dot(x_ref[...], w_ref[...].T) + b_ref[...]


def _crs_proj(xc2, p):
    BF = 400
    return pl.pallas_call(
        _crs_body,
        grid=(2000 // BF,),
        in_specs=[
            pl.BlockSpec((BF, 512), lambda i: (i, 0)),
            pl.BlockSpec((2048, 512), lambda i: (0, 0)),
            pl.BlockSpec((2048,), lambda i: (0,)),
        ],
        out_specs=pl.BlockSpec((BF, 2048), lambda i: (i, 0)),
        out_shape=jax.ShapeDtypeStruct((2000, 2048), F32),
    )(xc2, p['crs_w'], p['crs_b'])


def kernel(x_fine, x_coarse, x_global, edge_index_fine_coarse,
           edge_index_coarse_global, params):
    p = params
    B = x_global.shape[0]

    def prep(ei, n_src, n_dst):
        src, dst = ei[0], ei[1]
        n = min(n_src, n_dst)
        loops = jnp.arange(n, dtype=src.dtype)
        val = jnp.concatenate([(src != dst), jnp.ones((n,), bool)]).astype(F32)
        src = jnp.concatenate([src, loops])
        dst = jnp.concatenate([dst, loops])
        return src, dst, val

    src1, dst1, val1 = prep(edge_index_fine_coarse, 10000, 2000)
    src2, dst2, val2 = prep(edge_index_coarse_global, 2000, 20)
    padn = E2P - src2.shape[0]
    src2 = jnp.concatenate([src2, jnp.zeros((padn,), src2.dtype)])
    dst2 = jnp.concatenate([dst2, jnp.zeros((padn,), dst2.dtype)])
    val2 = jnp.concatenate([val2, jnp.zeros((padn,), F32)])

    xs, als = _fine_front(x_fine, p)
    ald = _coarse_front(x_coarse, p)

    M = jnp.max(als, axis=0) + jnp.max(ald, axis=0)
    M = jnp.where(M >= 0, M, 0.2 * M)

    U, den = _gat1_edges(xs, als, ald, src1, dst1, val1, M)

    ff = _fine_out(x_fine, p)

    xc2, xs2, als2 = _tail_a(U, den, p)
    xgout = _tail_b(x_global, xs2, als2, src2, dst2, val2, p)
    xcout = _crs_proj(xc2, p)

    return jnp.concatenate([
        xgout.reshape(B, 1, 2048),
        xcout.reshape(B, 100, 2048),
        ff.reshape(B, 500, 2048),
    ], axis=1)
